# baseline (speedup 1.0000x reference)
import functools

import jax
import jax.numpy as jnp
from jax import lax
from jax.experimental import pallas as pl
from jax.experimental.pallas import tpu as pltpu
from jax.experimental.pallas import tpu_sc as plsc

B = 16384
D = 8
NROWS = 1_000_000
NC = 2
NS = 16
NW = NC * NS
BPW = B // NW
CH = 128
NCH = BPW // CH

DCBLK = 2048
DGRID = (NROWS + DCBLK - 1) // DCBLK


def _detile_body(t0, t1, t2, t3, o0, o1, o2, o3):
    for tin, tout in ((t0, o0), (t1, o1), (t2, o2), (t3, o3)):
        tout[...] = tin[...].T


def _detile(guT, giT, muT, miT):
    return pl.pallas_call(
        _detile_body,
        grid=(DGRID,),
        in_specs=[pl.BlockSpec((D, DCBLK), lambda i: (0, i))] * 4,
        out_specs=[pl.BlockSpec((DCBLK, D), lambda i: (i, 0))] * 4,
        out_shape=[jax.ShapeDtypeStruct((NROWS, D), jnp.float32)] * 4,
    )(guT, giT, muT, miT)


@functools.lru_cache(maxsize=1)
def _make_sc_gather():
    mesh = plsc.VectorSubcoreMesh(core_axis_name="c", subcore_axis_name="s")

    @functools.partial(
        pl.kernel,
        mesh=mesh,
        compiler_params=pltpu.CompilerParams(use_tc_tiling_on_sc=False),
        out_type=[jax.ShapeDtypeStruct((B, D), jnp.float32) for _ in range(4)],
        scratch_types=[
            pltpu.VMEM((NCH, CH), jnp.int32),
            pltpu.VMEM((NCH, CH), jnp.int32),
            pltpu.VMEM((BPW, D), jnp.float32),
            pltpu.VMEM((BPW, D), jnp.float32),
            pltpu.VMEM((BPW, D), jnp.float32),
            pltpu.VMEM((BPW, D), jnp.float32),
            pltpu.SemaphoreType.DMA,
        ],
    )
    def _sc_gather(users_hbm, items_hbm, gu_hbm, gi_hbm, mu_hbm, mi_hbm,
                   o_gu, o_gi, o_mu, o_mi,
                   uidx, iidx, r_gu, r_gi, r_mu, r_mi, sem):
        wid = lax.axis_index("s") * NC + lax.axis_index("c")
        base = wid * BPW
        pltpu.sync_copy(users_hbm.at[pl.ds(wid * NCH, NCH)], uidx)
        pltpu.sync_copy(items_hbm.at[pl.ds(wid * NCH, NCH)], iidx)
        copies = []
        for k in range(NCH):
            sl = pl.ds(k * CH, CH)
            copies.append(pltpu.async_copy(gu_hbm.at[uidx.at[k]], r_gu.at[sl], sem))
            copies.append(pltpu.async_copy(gi_hbm.at[iidx.at[k]], r_gi.at[sl], sem))
            copies.append(pltpu.async_copy(mu_hbm.at[uidx.at[k]], r_mu.at[sl], sem))
            copies.append(pltpu.async_copy(mi_hbm.at[iidx.at[k]], r_mi.at[sl], sem))
        for cp in copies:
            cp.wait()
        out_sl = pl.ds(base, BPW)
        pltpu.sync_copy(r_gu, o_gu.at[out_sl])
        pltpu.sync_copy(r_gi, o_gi.at[out_sl])
        pltpu.sync_copy(r_mu, o_mu.at[out_sl])
        pltpu.sync_copy(r_mi, o_mi.at[out_sl])

    return _sc_gather


TC_BLK = 2048


def _tc_body(gu_ref, gi_ref, mu_ref, mi_ref,
             W0_ref, b0_ref, W1_ref, b1_ref, W2_ref, b2_ref, W3_ref, b3_ref,
             Wout_ref, bout_ref, out_ref):
    mu = mu_ref[...]
    mi = mi_ref[...]
    W0 = W0_ref[...]
    h = mu @ W0[:D, :] + mi @ W0[D:, :] + b0_ref[...]
    h = jnp.maximum(h, 0.0)
    h = jnp.maximum(h @ W1_ref[...] + b1_ref[...], 0.0)
    h = jnp.maximum(h @ W2_ref[...] + b2_ref[...], 0.0)
    h = jnp.maximum(h @ W3_ref[...] + b3_ref[...], 0.0)
    g = gu_ref[...] * gi_ref[...]
    Wout = Wout_ref[...]
    logit = h @ Wout[:D, :] + g @ Wout[D:, :] + bout_ref[...]
    out_ref[...] = jax.nn.sigmoid(logit)


def kernel(users, items, gmf_user_table, gmf_item_table, mlp_user_table,
           mlp_item_table, W0, b0, W1, b1, W2, b2, W3, b3, W_out, b_out):
    users_r = users.astype(jnp.int32).reshape(B // CH, CH)
    items_r = items.astype(jnp.int32).reshape(B // CH, CH)
    gu_d, gi_d, mu_d, mi_d = _detile(
        gmf_user_table.T, gmf_item_table.T, mlp_user_table.T,
        mlp_item_table.T)
    gu, gi, mu, mi = _make_sc_gather()(users_r, items_r,
                                       gu_d, gi_d, mu_d, mi_d)

    grid = B // TC_BLK
    data_spec = pl.BlockSpec((TC_BLK, D), lambda i: (i, 0))

    def wspec(shape):
        return pl.BlockSpec(shape, lambda i: tuple(0 for _ in shape))

    pred = pl.pallas_call(
        _tc_body,
        grid=(grid,),
        in_specs=[
            data_spec, data_spec, data_spec, data_spec,
            wspec(W0.shape), wspec((1, b0.shape[0])),
            wspec(W1.shape), wspec((1, b1.shape[0])),
            wspec(W2.shape), wspec((1, b2.shape[0])),
            wspec(W3.shape), wspec((1, b3.shape[0])),
            wspec(W_out.shape), wspec((1, 1)),
        ],
        out_specs=pl.BlockSpec((TC_BLK, 1), lambda i: (i, 0)),
        out_shape=jax.ShapeDtypeStruct((B, 1), jnp.float32),
    )(gu, gi, mu, mi,
      W0, b0.reshape(1, -1), W1, b1.reshape(1, -1), W2, b2.reshape(1, -1),
      W3, b3.reshape(1, -1), W_out, b_out.reshape(1, 1))
    return pred
